# baseline (speedup 1.0000x reference)
import functools

import jax
import jax.numpy as jnp
from jax.experimental import pallas as pl
from jax.experimental.pallas import tpu as pltpu
from jax.experimental.pallas import tpu_sc as plsc

_N = 20000
_NC = 18
_C = 128
_NCROSS = 2048
_K = 256
_L = 128
_R = 160
_NPAD = _R * _L
_BIG = 2 ** 30

_SC_NC = 2
_SC_NS = 16
_SC_NW = _SC_NC * _SC_NS


def _sigmoid(x):
    return 1.0 / (1.0 + jnp.exp(-x))


def _iota3():
    idx2d = (jax.lax.broadcasted_iota(jnp.int32, (_R, _L), 0) * _L
             + jax.lax.broadcasted_iota(jnp.int32, (_R, _L), 1))
    return idx2d.reshape(_R // 8, 8, _L)


def _fold_ew(x, op, stop):
    w = x.shape[0]
    while w > stop:
        h = w // 2
        comb = op(x[0:h], x[h:2 * h])
        if w % 2:
            x = jnp.concatenate([comb, x[2 * h:w]], axis=0)
            w = h + 1
        else:
            x = comb
            w = h
    return x


def _slane_bfly_ew(x, op):
    for sh in (4, 2, 1):
        x = op(x, pltpu.roll(x, sh, 0))
    return x


def _argmax_first(val, idxp):
    v8 = _slane_bfly_ew(
        _fold_ew(val, jnp.maximum, 1).reshape(8, _L), jnp.maximum)
    cand = jnp.where(val == v8[None], idxp, _BIG)
    i8 = _slane_bfly_ew(
        _fold_ew(cand, jnp.minimum, 1).reshape(8, _L), jnp.minimum)
    vt = jnp.swapaxes(v8, 0, 1)
    it = jnp.swapaxes(i8, 0, 1)
    mt = _slane_bfly_ew(_fold_ew(vt, jnp.maximum, 8), jnp.maximum)
    candt = jnp.where(vt == mt[0:1], it, _BIG)
    imin = _slane_bfly_ew(_fold_ew(candt, jnp.minimum, 8), jnp.minimum)
    return imin[0, 0]


def _argmin_idx(cand):
    i8 = _slane_bfly_ew(
        _fold_ew(cand, jnp.minimum, 1).reshape(8, _L), jnp.minimum)
    it = jnp.swapaxes(i8, 0, 1)
    imin = _slane_bfly_ew(_fold_ew(it, jnp.minimum, 8), jnp.minimum)
    return imin[0, 0]


def _main_kernel(stk_ref, feat_ref, psm_ref,
                 pts_out, feats_out, inds_out, cinds_out,
                 mask_ref, bits_ref):
    idx3 = _iota3()
    idx2d = idx3.reshape(_R, _L)
    valid = idx2d < _N

    sig_c = _sigmoid(stk_ref[0])
    m = jnp.full((_R, _L), -1.0, dtype=jnp.float32)
    for c in range(_NC):
        m = jnp.maximum(m, _sigmoid(stk_ref[1 + c]) * sig_c)
    scores = jnp.where(valid, m, 0.0)
    bits_ref[...] = jax.lax.bitcast_convert_type(scores, jnp.int32)

    def bs_body(_, lohi):
        lo, hi = lohi
        mid = jax.lax.div(lo + hi, jnp.int32(2))
        cnt = jnp.sum(jnp.where(bits_ref[...] >= mid, 1, 0))
        big = cnt >= _K
        return (jnp.where(big, mid, lo), jnp.where(big, hi, mid))

    lo0 = jnp.int32(1)
    hi0 = jnp.int32(0x3F800001)
    t, _ = jax.lax.fori_loop(0, 31, bs_body, (lo0, hi0))

    bits = bits_ref[...]
    gt = bits > t
    eq = bits == t
    cnt_gt = jnp.sum(jnp.where(gt, 1, 0))
    quota = _K - cnt_gt

    x = jnp.where(eq, 1, 0)
    eqi = x
    for sh in (1, 2, 4, 8, 16, 32, 64):
        x = x + jnp.pad(x, ((0, 0), (sh, 0)))[:, :_L]
    lane_inc = x
    row_tot = lane_inc[:, _L - 1:_L]
    y = row_tot
    for sh in (1, 2, 4, 8, 16, 32, 64, 128):
        y = y + jnp.pad(y, ((sh, 0), (0, 0)))[:_R, :]
    rank = (y - row_tot) + (lane_inc - eqi)
    selected = gt | (eq & (rank < quota))
    mask_ref[...] = jnp.where(selected, 1, 0).reshape(_R // 8, 8, _L)

    def ext_body(j, carry):
        msk = mask_ref[...]
        cand = jnp.where(msk != 0, idx3, _BIG)
        idx = _argmin_idx(cand)
        mask_ref[...] = jnp.where(idx3 == idx, 0, msk)
        inds_out[j] = idx
        feats_out[pl.ds(j, 1), :] = feat_ref[pl.ds(idx, 1), :]
        base = idx * 3
        pts_out[pl.ds(j, 1), :] = jnp.concatenate(
            [psm_ref[base].reshape(1, 1), psm_ref[base + 1].reshape(1, 1),
             psm_ref[base + 2].reshape(1, 1)], axis=1)
        return carry

    jax.lax.fori_loop(0, _K, ext_body, 0)

    px3 = stk_ref[1 + _NC].reshape(_R // 8, 8, _L)
    py3 = stk_ref[2 + _NC].reshape(_R // 8, 8, _L)
    pz3 = stk_ref[3 + _NC].reshape(_R // 8, 8, _L)

    cinds_out[0] = jnp.int32(0)
    lx0 = psm_ref[0]
    ly0 = psm_ref[1]
    lz0 = psm_ref[2]
    d0 = jnp.where(idx3 < _N, jnp.inf, -jnp.inf)

    def fps_iter(i, lx, ly, lz, dists):
        dx = px3 - lx
        dy = py3 - ly
        dz = pz3 - lz
        d = (dx * dx + dy * dy) + dz * dz
        nd = jnp.minimum(dists, d)
        nxt = _argmax_first(nd, idx3)
        cinds_out[i] = nxt
        base = nxt * 3
        return (psm_ref[base], psm_ref[base + 1], psm_ref[base + 2], nd)

    def fps_pair(k, carry):
        i1 = 2 * k + 1
        s1 = fps_iter(i1, *carry)
        return fps_iter(i1 + 1, *s1)

    carry = jax.lax.fori_loop(0, (_NCROSS - 2) // 2, fps_pair,
                              (lx0, ly0, lz0, d0))
    fps_iter(_NCROSS - 1, *carry)


def _sc_gather_call(table, idx):
    bpw = _NCROSS // _SC_NW
    mesh = plsc.VectorSubcoreMesh(core_axis_name="c", subcore_axis_name="s")

    @functools.partial(
        pl.kernel, mesh=mesh,
        out_type=jax.ShapeDtypeStruct((_NCROSS, _C), jnp.float32),
        scratch_types=[
            pltpu.VMEM((bpw,), jnp.int32),
            pltpu.VMEM((bpw, _C), jnp.float32),
            pltpu.SemaphoreType.DMA,
        ],
    )
    def k(table_hbm, idx_hbm, out_hbm, idx_v, rows_v, sem):
        wid = jax.lax.axis_index("s") * _SC_NC + jax.lax.axis_index("c")
        base = wid * bpw
        pltpu.sync_copy(idx_hbm.at[pl.ds(base, bpw)], idx_v)
        pltpu.async_copy(table_hbm.at[idx_v], rows_v, sem).wait()
        pltpu.sync_copy(rows_v, out_hbm.at[pl.ds(base, bpw)])

    return k(table, idx)


def kernel(centernesses, cls_scores, points, features):
    pad = _NPAD - _N
    stack = jnp.concatenate(
        [centernesses[None, :], cls_scores.T, points.T], axis=0)
    stack = jnp.pad(stack, ((0, 0), (0, pad))).reshape(4 + _NC, _R, _L)
    psm = points.reshape(_N * 3)

    pts, feats, inds, cinds = pl.pallas_call(
        _main_kernel,
        in_specs=[
            pl.BlockSpec(memory_space=pltpu.VMEM),
            pl.BlockSpec(memory_space=pltpu.VMEM),
            pl.BlockSpec(memory_space=pltpu.SMEM),
        ],
        out_shape=[
            jax.ShapeDtypeStruct((_K, 3), jnp.float32),
            jax.ShapeDtypeStruct((_K, _C), jnp.float32),
            jax.ShapeDtypeStruct((_K,), jnp.int32),
            jax.ShapeDtypeStruct((_NCROSS,), jnp.int32),
        ],
        out_specs=[
            pl.BlockSpec(memory_space=pltpu.VMEM),
            pl.BlockSpec(memory_space=pltpu.VMEM),
            pl.BlockSpec(memory_space=pltpu.SMEM),
            pl.BlockSpec(memory_space=pltpu.SMEM),
        ],
        scratch_shapes=[
            pltpu.VMEM((_R // 8, 8, _L), jnp.int32),
            pltpu.VMEM((_R, _L), jnp.int32),
        ],
    )(stack, features, psm)

    cross = _sc_gather_call(features, cinds)
    return (pts[None], feats[None], inds[None], cross[None])


# speedup vs baseline: 1.0329x; 1.0329x over previous
import functools

import jax
import jax.numpy as jnp
from jax.experimental import pallas as pl
from jax.experimental.pallas import tpu as pltpu
from jax.experimental.pallas import tpu_sc as plsc

_N = 20000
_NC = 18
_C = 128
_NCROSS = 2048
_K = 256
_L = 128
_R = 160
_NPAD = _R * _L
_BIG = 2 ** 30

_SC_NC = 2
_SC_NS = 16
_SC_NW = _SC_NC * _SC_NS


def _sigmoid(x):
    return 1.0 / (1.0 + jnp.exp(-x))


def _iota3():
    idx2d = (jax.lax.broadcasted_iota(jnp.int32, (_R, _L), 0) * _L
             + jax.lax.broadcasted_iota(jnp.int32, (_R, _L), 1))
    return idx2d.reshape(_R // 8, 8, _L)


def _fold_ew(x, op, stop):
    w = x.shape[0]
    while w > stop:
        h = w // 2
        comb = op(x[0:h], x[h:2 * h])
        if w % 2:
            x = jnp.concatenate([comb, x[2 * h:w]], axis=0)
            w = h + 1
        else:
            x = comb
            w = h
    return x


def _slane_bfly_ew(x, op):
    for sh in (4, 2, 1):
        x = op(x, pltpu.roll(x, sh, 0))
    return x


def _argmax_first(val, idxp):
    v8 = _slane_bfly_ew(
        _fold_ew(val, jnp.maximum, 1).reshape(8, _L), jnp.maximum)
    cand = jnp.where(val == v8[None], idxp, _BIG)
    i8 = _slane_bfly_ew(
        _fold_ew(cand, jnp.minimum, 1).reshape(8, _L), jnp.minimum)
    vt = jnp.swapaxes(v8, 0, 1)
    it = jnp.swapaxes(i8, 0, 1)
    mt = _slane_bfly_ew(_fold_ew(vt, jnp.maximum, 8), jnp.maximum)
    candt = jnp.where(vt == mt[0:1], it, _BIG)
    imin = _slane_bfly_ew(_fold_ew(candt, jnp.minimum, 8), jnp.minimum)
    return imin[0, 0]


def _argmin2_idx(cand):
    def merge(a, b):
        a1, a2 = a
        b1, b2 = b
        lo = jnp.minimum(a1, b1)
        hi = jnp.maximum(a1, b1)
        return (lo, jnp.minimum(hi, jnp.minimum(a2, b2)))

    def fold(pair, stop):
        w = pair[0].shape[0]
        while w > stop:
            h = w // 2
            comb = merge(tuple(p[0:h] for p in pair),
                         tuple(p[h:2 * h] for p in pair))
            if w % 2:
                pair = tuple(jnp.concatenate([pc, p[2 * h:w]], axis=0)
                             for pc, p in zip(comb, pair))
                w = h + 1
            else:
                pair = comb
                w = h
        return pair

    def bfly(pair):
        for sh in (4, 2, 1):
            pair = merge(tuple(pltpu.roll(p, sh, 0) for p in pair), pair)
        return pair

    pair = (cand, jnp.full(cand.shape, _BIG, dtype=jnp.int32))
    pair = bfly(tuple(p.reshape(8, _L) for p in fold(pair, 1)))
    pair = bfly(fold(tuple(jnp.swapaxes(p, 0, 1) for p in pair), 8))
    return pair[0][0, 0], pair[1][0, 0]


def _main_kernel(stk_ref, feat_ref, psm_ref,
                 pts_out, feats_out, inds_out, cinds_out,
                 mask_ref, bits_ref):
    idx3 = _iota3()
    idx2d = idx3.reshape(_R, _L)
    valid = idx2d < _N

    sig_c = _sigmoid(stk_ref[0])
    m = jnp.full((_R, _L), -1.0, dtype=jnp.float32)
    for c in range(_NC):
        m = jnp.maximum(m, _sigmoid(stk_ref[1 + c]) * sig_c)
    scores = jnp.where(valid, m, 0.0)
    bits_ref[...] = jax.lax.bitcast_convert_type(scores, jnp.int32)

    def bs_body(_, lohi):
        lo, hi = lohi
        mid = jax.lax.div(lo + hi, jnp.int32(2))
        cnt = jnp.sum(jnp.where(bits_ref[...] >= mid, 1, 0))
        big = cnt >= _K
        return (jnp.where(big, mid, lo), jnp.where(big, hi, mid))

    lo0 = jnp.int32(1)
    hi0 = jnp.int32(0x3F800001)
    t, _ = jax.lax.fori_loop(0, 31, bs_body, (lo0, hi0))

    bits = bits_ref[...]
    gt = bits > t
    eq = bits == t
    cnt_gt = jnp.sum(jnp.where(gt, 1, 0))
    quota = _K - cnt_gt

    x = jnp.where(eq, 1, 0)
    eqi = x
    for sh in (1, 2, 4, 8, 16, 32, 64):
        x = x + jnp.pad(x, ((0, 0), (sh, 0)))[:, :_L]
    lane_inc = x
    row_tot = lane_inc[:, _L - 1:_L]
    y = row_tot
    for sh in (1, 2, 4, 8, 16, 32, 64, 128):
        y = y + jnp.pad(y, ((sh, 0), (0, 0)))[:_R, :]
    rank = (y - row_tot) + (lane_inc - eqi)
    selected = gt | (eq & (rank < quota))
    mask_ref[...] = jnp.where(selected, 1, 0).reshape(_R // 8, 8, _L)

    def ext_body(j, carry):
        msk = mask_ref[...]
        cand = jnp.where(msk != 0, idx3, _BIG)
        i1, i2 = _argmin2_idx(cand)
        mask_ref[...] = jnp.where((idx3 == i1) | (idx3 == i2), 0, msk)
        for q, idx in ((2 * j, i1), (2 * j + 1, i2)):
            inds_out[q] = idx
            feats_out[pl.ds(q, 1), :] = feat_ref[pl.ds(idx, 1), :]
            base = idx * 3
            pts_out[pl.ds(q, 1), :] = jnp.concatenate(
                [psm_ref[base].reshape(1, 1),
                 psm_ref[base + 1].reshape(1, 1),
                 psm_ref[base + 2].reshape(1, 1)], axis=1)
        return carry

    jax.lax.fori_loop(0, _K // 2, ext_body, 0)

    px3 = stk_ref[1 + _NC].reshape(_R // 8, 8, _L)
    py3 = stk_ref[2 + _NC].reshape(_R // 8, 8, _L)
    pz3 = stk_ref[3 + _NC].reshape(_R // 8, 8, _L)

    cinds_out[0] = jnp.int32(0)
    lx0 = psm_ref[0]
    ly0 = psm_ref[1]
    lz0 = psm_ref[2]
    d0 = jnp.where(idx3 < _N, jnp.inf, -jnp.inf)

    def fps_iter(i, lx, ly, lz, dists):
        dx = px3 - lx
        dy = py3 - ly
        dz = pz3 - lz
        d = (dx * dx + dy * dy) + dz * dz
        nd = jnp.minimum(dists, d)
        nxt = _argmax_first(nd, idx3)
        cinds_out[i] = nxt
        base = nxt * 3
        return (psm_ref[base], psm_ref[base + 1], psm_ref[base + 2], nd)

    def fps_pair(k, carry):
        i1 = 2 * k + 1
        s1 = fps_iter(i1, *carry)
        return fps_iter(i1 + 1, *s1)

    carry = jax.lax.fori_loop(0, (_NCROSS - 2) // 2, fps_pair,
                              (lx0, ly0, lz0, d0))
    fps_iter(_NCROSS - 1, *carry)


def _sc_gather_call(table, idx):
    bpw = _NCROSS // _SC_NW
    mesh = plsc.VectorSubcoreMesh(core_axis_name="c", subcore_axis_name="s")

    @functools.partial(
        pl.kernel, mesh=mesh,
        out_type=jax.ShapeDtypeStruct((_NCROSS, _C), jnp.float32),
        scratch_types=[
            pltpu.VMEM((bpw,), jnp.int32),
            pltpu.VMEM((bpw, _C), jnp.float32),
            pltpu.SemaphoreType.DMA,
        ],
    )
    def k(table_hbm, idx_hbm, out_hbm, idx_v, rows_v, sem):
        wid = jax.lax.axis_index("s") * _SC_NC + jax.lax.axis_index("c")
        base = wid * bpw
        pltpu.sync_copy(idx_hbm.at[pl.ds(base, bpw)], idx_v)
        pltpu.async_copy(table_hbm.at[idx_v], rows_v, sem).wait()
        pltpu.sync_copy(rows_v, out_hbm.at[pl.ds(base, bpw)])

    return k(table, idx)


def kernel(centernesses, cls_scores, points, features):
    pad = _NPAD - _N
    stack = jnp.concatenate(
        [centernesses[None, :], cls_scores.T, points.T], axis=0)
    stack = jnp.pad(stack, ((0, 0), (0, pad))).reshape(4 + _NC, _R, _L)
    psm = points.reshape(_N * 3)

    pts, feats, inds, cinds = pl.pallas_call(
        _main_kernel,
        in_specs=[
            pl.BlockSpec(memory_space=pltpu.VMEM),
            pl.BlockSpec(memory_space=pltpu.VMEM),
            pl.BlockSpec(memory_space=pltpu.SMEM),
        ],
        out_shape=[
            jax.ShapeDtypeStruct((_K, 3), jnp.float32),
            jax.ShapeDtypeStruct((_K, _C), jnp.float32),
            jax.ShapeDtypeStruct((_K,), jnp.int32),
            jax.ShapeDtypeStruct((_NCROSS,), jnp.int32),
        ],
        out_specs=[
            pl.BlockSpec(memory_space=pltpu.VMEM),
            pl.BlockSpec(memory_space=pltpu.VMEM),
            pl.BlockSpec(memory_space=pltpu.SMEM),
            pl.BlockSpec(memory_space=pltpu.SMEM),
        ],
        scratch_shapes=[
            pltpu.VMEM((_R // 8, 8, _L), jnp.int32),
            pltpu.VMEM((_R, _L), jnp.int32),
        ],
    )(stack, features, psm)

    cross = _sc_gather_call(features, cinds)
    return (pts[None], feats[None], inds[None], cross[None])
